# baseline (speedup 1.0000x reference)
import functools

import jax
import jax.numpy as jnp
from jax import lax
from jax.experimental import pallas as pl
from jax.experimental.pallas import tpu as pltpu
from jax.experimental.pallas import tpu_sc as plsc

_NUM_BUCKETS = 1024
_NUM_HEADS = 4
_EMBED_DIM = 256
_HEAD_DIM = 64
_COMMITMENT_COST = 0.25
_EPSILON = 1e-5
_B, _T = 32, 1024
_N = _B * _T
_R = 1024
_G = _N // _R
_KDIM = _NUM_HEADS * _NUM_BUCKETS
_NROWS = _N * _NUM_HEADS

_NW = 32
_BPW = _NROWS // _NW
_CH = 1024


def _vq_tc_kernel(x_ref, w_ref, b_ref, e_ref, e2_ref,
                  idx_ref, loss_ref, uniq_ref,
                  counts_acc, sxn2_acc, smax_acc):
    step = pl.program_id(0)

    @pl.when(step == 0)
    def _init():
        counts_acc[...] = jnp.zeros_like(counts_acc)
        sxn2_acc[...] = jnp.zeros_like(sxn2_acc)
        smax_acc[...] = jnp.zeros_like(smax_acc)

    x = x_ref[...]
    mu = jnp.mean(x, axis=-1, keepdims=True)
    var = jnp.mean((x - mu) ** 2, axis=-1, keepdims=True)
    x = (x - mu) / jnp.sqrt(var + 1e-5) * w_ref[...] + b_ref[...]
    x = jnp.tanh(x / 5.0) * 5.0
    n = jnp.sqrt(jnp.sum(x * x, axis=-1, keepdims=True))
    xn = x / jnp.maximum(n, _EPSILON)
    sxn2_acc[...] += jnp.sum(xn * xn, axis=(0, 1), keepdims=True)

    dist = jnp.dot(xn, e_ref[...], preferred_element_type=jnp.float32)

    ihs = []
    for h in range(_NUM_HEADS):
        dh = dist[:, h * _NUM_BUCKETS:(h + 1) * _NUM_BUCKETS]
        mh = jnp.max(dh, axis=-1, keepdims=True)
        smax_acc[...] += jnp.sum(mh, axis=(0, 1), keepdims=True)
        mask = (dh >= mh).astype(jnp.float32)
        counts_acc[:, h * _NUM_BUCKETS:(h + 1) * _NUM_BUCKETS] += (
            jnp.sum(mask, axis=0, keepdims=True))
        iota = lax.broadcasted_iota(jnp.int32, dh.shape, 1)
        cand = jnp.where(dh >= mh, iota, _NUM_BUCKETS)
        ihs.append(jnp.min(cand, axis=-1, keepdims=True) +
                   (h * _NUM_BUCKETS))
    idx_ref[...] = jnp.concatenate(ihs, axis=-1)

    @pl.when(step == _G - 1)
    def _fin():
        sq = e2_ref[...] * e2_ref[...]
        v = jnp.dot(counts_acc[...], sq,
                    preferred_element_type=jnp.float32)
        sq2 = jnp.sum(v, axis=(0, 1), keepdims=True)
        num = sxn2_acc[...] + sq2 - 2.0 * smax_acc[...]
        loss_ref[...] = (_COMMITMENT_COST / (_N * _EMBED_DIM)) * num
        c = counts_acc[0:1, 0:_NUM_BUCKETS]
        for h in range(1, _NUM_HEADS):
            c = c + counts_acc[0:1, h * _NUM_BUCKETS:(h + 1) * _NUM_BUCKETS]
        uniq_ref[...] = jnp.sum((c > 0.5).astype(jnp.int32),
                                axis=(0, 1), keepdims=True)


def _tc_stage(x, w, b, e, e2):
    return pl.pallas_call(
        _vq_tc_kernel,
        grid=(_G,),
        in_specs=[
            pl.BlockSpec((_R, _EMBED_DIM), lambda i: (i, 0)),
            pl.BlockSpec((1, _EMBED_DIM), lambda i: (0, 0)),
            pl.BlockSpec((1, _EMBED_DIM), lambda i: (0, 0)),
            pl.BlockSpec((_EMBED_DIM, _KDIM), lambda i: (0, 0)),
            pl.BlockSpec((_KDIM, _EMBED_DIM), lambda i: (0, 0)),
        ],
        out_specs=[
            pl.BlockSpec((_R, _NUM_HEADS), lambda i: (i, 0)),
            pl.BlockSpec((1, 1), lambda i: (0, 0)),
            pl.BlockSpec((1, 1), lambda i: (0, 0)),
        ],
        out_shape=[
            jax.ShapeDtypeStruct((_N, _NUM_HEADS), jnp.int32),
            jax.ShapeDtypeStruct((1, 1), jnp.float32),
            jax.ShapeDtypeStruct((1, 1), jnp.int32),
        ],
        scratch_shapes=[
            pltpu.VMEM((1, _KDIM), jnp.float32),
            pltpu.VMEM((1, 1), jnp.float32),
            pltpu.VMEM((1, 1), jnp.float32),
        ],
    )(x, w, b, e, e2)


def _sc_gather(table, gidx):
    mesh = plsc.VectorSubcoreMesh(core_axis_name="c", subcore_axis_name="s")

    @functools.partial(
        pl.kernel,
        mesh=mesh,
        compiler_params=pltpu.CompilerParams(use_tc_tiling_on_sc=False),
        out_type=jax.ShapeDtypeStruct((_NROWS, _HEAD_DIM), jnp.float32),
        scratch_types=[
            pltpu.VMEM((_CH,), jnp.int32),
            pltpu.VMEM((_CH, _HEAD_DIM), jnp.float32),
            pltpu.SemaphoreType.DMA,
        ],
    )
    def k(table_hbm, idx_hbm, out_hbm, idx_v, rows_v, sem):
        wid = lax.axis_index("s") * 2 + lax.axis_index("c")
        base = wid * _BPW

        @pl.loop(0, _BPW, step=_CH)
        def _(off):
            pltpu.sync_copy(idx_hbm.at[pl.ds(base + off, _CH)], idx_v)
            pltpu.async_copy(table_hbm.at[idx_v], rows_v, sem).wait()
            pltpu.sync_copy(rows_v, out_hbm.at[pl.ds(base + off, _CH)])

    return k(table, gidx)


@functools.partial(jax.jit, static_argnames=())
def kernel(inputs, ln_weight, ln_bias, embeddings):
    x = inputs.reshape(_N, _EMBED_DIM)
    w = ln_weight.reshape(1, _EMBED_DIM)
    b = ln_bias.reshape(1, _EMBED_DIM)

    eT = jnp.transpose(embeddings, (0, 2, 1))
    e_blocks = []
    e2_blocks = []
    for h in range(_NUM_HEADS):
        row = [jnp.zeros((_HEAD_DIM, _NUM_BUCKETS), jnp.float32)] * _NUM_HEADS
        row[h] = eT[h]
        e_blocks.append(jnp.concatenate(row, axis=-1))
        row2 = [jnp.zeros((_NUM_BUCKETS, _HEAD_DIM), jnp.float32)] * _NUM_HEADS
        row2[h] = embeddings[h]
        e2_blocks.append(jnp.concatenate(row2, axis=-1))
    e = jnp.concatenate(e_blocks, axis=0)
    e2 = jnp.concatenate(e2_blocks, axis=0)
    table = embeddings.reshape(_KDIM, _HEAD_DIM)

    gidx, loss, uniq = _tc_stage(x, w, b, e, e2)
    q = _sc_gather(table, gidx.reshape(_NROWS))

    quantized_st = q.reshape(_B, _T, _EMBED_DIM)
    return (quantized_st, loss.reshape(()), uniq.reshape(()))


# speedup vs baseline: 1.0541x; 1.0541x over previous
import functools

import jax
import jax.numpy as jnp
from jax import lax
from jax.experimental import pallas as pl
from jax.experimental.pallas import tpu as pltpu
from jax.experimental.pallas import tpu_sc as plsc

_NUM_BUCKETS = 1024
_NUM_HEADS = 4
_EMBED_DIM = 256
_HEAD_DIM = 64
_COMMITMENT_COST = 0.25
_EPSILON = 1e-5
_B, _T = 32, 1024
_N = _B * _T
_R = 1024
_G = _N // _R
_KDIM = _NUM_HEADS * _NUM_BUCKETS
_NROWS = _N * _NUM_HEADS

_NW = 32
_BPW = _NROWS // _NW
_CH = 1024


def _vq_tc_kernel(x_ref, w_ref, b_ref, et_ref,
                  idx_ref, loss_ref, uniq_ref,
                  e_s, counts_acc, sxn2_acc, smax_acc):
    step = pl.program_id(0)

    @pl.when(step == 0)
    def _init():
        counts_acc[...] = jnp.zeros_like(counts_acc)
        sxn2_acc[...] = jnp.zeros_like(sxn2_acc)
        smax_acc[...] = jnp.zeros_like(smax_acc)
        e_s[...] = jnp.zeros_like(e_s)
        for h in range(_NUM_HEADS):
            e_s[h * _HEAD_DIM:(h + 1) * _HEAD_DIM,
                h * _NUM_BUCKETS:(h + 1) * _NUM_BUCKETS] = et_ref[h]

    x = x_ref[...]
    mu = jnp.mean(x, axis=-1, keepdims=True)
    var = jnp.mean((x - mu) ** 2, axis=-1, keepdims=True)
    x = (x - mu) / jnp.sqrt(var + 1e-5) * w_ref[...] + b_ref[...]
    x = jnp.tanh(x / 5.0) * 5.0
    n = jnp.sqrt(jnp.sum(x * x, axis=-1, keepdims=True))
    xn = x / jnp.maximum(n, _EPSILON)
    sxn2_acc[...] += jnp.sum(xn * xn, axis=(0, 1), keepdims=True)

    dist = jnp.dot(xn, e_s[...], preferred_element_type=jnp.float32)

    ihs = []
    for h in range(_NUM_HEADS):
        dh = dist[:, h * _NUM_BUCKETS:(h + 1) * _NUM_BUCKETS]
        mh = jnp.max(dh, axis=-1, keepdims=True)
        smax_acc[...] += jnp.sum(mh, axis=(0, 1), keepdims=True)
        mask = (dh >= mh).astype(jnp.float32)
        counts_acc[:, h * _NUM_BUCKETS:(h + 1) * _NUM_BUCKETS] += (
            jnp.sum(mask, axis=0, keepdims=True))
        iota = lax.broadcasted_iota(jnp.int32, dh.shape, 1)
        cand = jnp.where(dh >= mh, iota, _NUM_BUCKETS)
        ihs.append(jnp.min(cand, axis=-1, keepdims=True) +
                   (h * _NUM_BUCKETS))
    idx_ref[...] = jnp.concatenate(ihs, axis=-1)

    @pl.when(step == _G - 1)
    def _fin():
        normsq = jnp.sum(e_s[...] * e_s[...], axis=0, keepdims=True)
        sq2 = jnp.sum(counts_acc[...] * normsq, axis=(0, 1), keepdims=True)
        num = sxn2_acc[...] + sq2 - 2.0 * smax_acc[...]
        loss_ref[...] = (_COMMITMENT_COST / (_N * _EMBED_DIM)) * num
        c = counts_acc[0:1, 0:_NUM_BUCKETS]
        for h in range(1, _NUM_HEADS):
            c = c + counts_acc[0:1, h * _NUM_BUCKETS:(h + 1) * _NUM_BUCKETS]
        uniq_ref[...] = jnp.sum((c > 0.5).astype(jnp.int32),
                                axis=(0, 1), keepdims=True)


def _tc_stage(x, w, b, et):
    return pl.pallas_call(
        _vq_tc_kernel,
        grid=(_G,),
        in_specs=[
            pl.BlockSpec((_R, _EMBED_DIM), lambda i: (i, 0)),
            pl.BlockSpec((1, _EMBED_DIM), lambda i: (0, 0)),
            pl.BlockSpec((1, _EMBED_DIM), lambda i: (0, 0)),
            pl.BlockSpec((_NUM_HEADS, _HEAD_DIM, _NUM_BUCKETS),
                         lambda i: (0, 0, 0)),
        ],
        out_specs=[
            pl.BlockSpec((_R, _NUM_HEADS), lambda i: (i, 0)),
            pl.BlockSpec((1, 1), lambda i: (0, 0)),
            pl.BlockSpec((1, 1), lambda i: (0, 0)),
        ],
        out_shape=[
            jax.ShapeDtypeStruct((_N, _NUM_HEADS), jnp.int32),
            jax.ShapeDtypeStruct((1, 1), jnp.float32),
            jax.ShapeDtypeStruct((1, 1), jnp.int32),
        ],
        scratch_shapes=[
            pltpu.VMEM((_EMBED_DIM, _KDIM), jnp.float32),
            pltpu.VMEM((1, _KDIM), jnp.float32),
            pltpu.VMEM((1, 1), jnp.float32),
            pltpu.VMEM((1, 1), jnp.float32),
        ],
    )(x, w, b, et)


def _sc_gather(table, gidx):
    mesh = plsc.VectorSubcoreMesh(core_axis_name="c", subcore_axis_name="s")

    @functools.partial(
        pl.kernel,
        mesh=mesh,
        compiler_params=pltpu.CompilerParams(use_tc_tiling_on_sc=False),
        out_type=jax.ShapeDtypeStruct((_NROWS, _HEAD_DIM), jnp.float32),
        scratch_types=[
            pltpu.VMEM((_CH,), jnp.int32),
            pltpu.VMEM((_CH, _HEAD_DIM), jnp.float32),
            pltpu.SemaphoreType.DMA,
        ],
    )
    def k(table_hbm, idx_hbm, out_hbm, idx_v, rows_v, sem):
        wid = lax.axis_index("s") * 2 + lax.axis_index("c")
        base = wid * _BPW

        @pl.loop(0, _BPW, step=_CH)
        def _(off):
            pltpu.sync_copy(idx_hbm.at[pl.ds(base + off, _CH)], idx_v)
            pltpu.async_copy(table_hbm.at[idx_v], rows_v, sem).wait()
            pltpu.sync_copy(rows_v, out_hbm.at[pl.ds(base + off, _CH)])

    return k(table, gidx)


@functools.partial(jax.jit, static_argnames=())
def kernel(inputs, ln_weight, ln_bias, embeddings):
    x = inputs.reshape(_N, _EMBED_DIM)
    w = ln_weight.reshape(1, _EMBED_DIM)
    b = ln_bias.reshape(1, _EMBED_DIM)
    et = jnp.transpose(embeddings, (0, 2, 1))
    table = embeddings.reshape(_KDIM, _HEAD_DIM)

    gidx, loss, uniq = _tc_stage(x, w, b, et)
    q = _sc_gather(table, gidx.reshape(_NROWS))

    quantized_st = q.reshape(_B, _T, _EMBED_DIM)
    return (quantized_st, loss.reshape(()), uniq.reshape(()))


# speedup vs baseline: 1.4063x; 1.3341x over previous
import functools

import jax
import jax.numpy as jnp
from jax.experimental import pallas as pl
from jax.experimental.pallas import tpu as pltpu

_NUM_BUCKETS = 1024
_NUM_HEADS = 4
_EMBED_DIM = 256
_HEAD_DIM = 64
_COMMITMENT_COST = 0.25
_EPSILON = 1e-5
_B, _T = 32, 1024
_N = _B * _T
_R = 1024
_G = _N // _R
_KDIM = _NUM_HEADS * _NUM_BUCKETS


def _vq_kernel(x_ref, w_ref, b_ref, e_ref, e2_ref,
               out_ref, loss_ref, uniq_ref,
               counts_acc, loss_acc):
    step = pl.program_id(0)

    @pl.when(step == 0)
    def _init():
        counts_acc[...] = jnp.zeros_like(counts_acc)
        loss_acc[...] = jnp.zeros_like(loss_acc)

    x = x_ref[...]
    mu = jnp.mean(x, axis=-1, keepdims=True)
    var = jnp.mean((x - mu) ** 2, axis=-1, keepdims=True)
    x = (x - mu) / jnp.sqrt(var + 1e-5) * w_ref[...] + b_ref[...]
    x = jnp.tanh(x / 5.0) * 5.0
    n = jnp.sqrt(jnp.sum(x * x, axis=-1, keepdims=True))
    xn = x / jnp.maximum(n, _EPSILON)

    dist = jnp.dot(xn, e_ref[...], preferred_element_type=jnp.float32)

    masks = []
    for h in range(_NUM_HEADS):
        dh = dist[:, h * _NUM_BUCKETS:(h + 1) * _NUM_BUCKETS]
        mh = jnp.max(dh, axis=-1, keepdims=True)
        masks.append((dh >= mh).astype(jnp.bfloat16))
    mask = jnp.concatenate(masks, axis=-1)

    counts_acc[...] += jnp.sum(mask.astype(jnp.float32), axis=0,
                               keepdims=True)

    q = jnp.dot(mask, e2_ref[...], preferred_element_type=jnp.float32)
    out_ref[...] = q

    diff = q - xn
    loss_acc[...] += jnp.sum(diff * diff, axis=(0, 1), keepdims=True)

    @pl.when(step == _G - 1)
    def _fin():
        loss_ref[...] = (_COMMITMENT_COST / (_N * _EMBED_DIM)) * loss_acc[...]
        c = counts_acc[0:1, 0:_NUM_BUCKETS]
        for h in range(1, _NUM_HEADS):
            c = c + counts_acc[0:1, h * _NUM_BUCKETS:(h + 1) * _NUM_BUCKETS]
        uniq_ref[...] = jnp.sum((c > 0.5).astype(jnp.int32),
                                axis=(0, 1), keepdims=True)


@functools.partial(jax.jit, static_argnames=())
def kernel(inputs, ln_weight, ln_bias, embeddings):
    x = inputs.reshape(_N, _EMBED_DIM)
    w = ln_weight.reshape(1, _EMBED_DIM)
    b = ln_bias.reshape(1, _EMBED_DIM)
    eT = jnp.transpose(embeddings, (0, 2, 1))
    e_blocks = []
    for h in range(_NUM_HEADS):
        row = [jnp.zeros((_HEAD_DIM, _NUM_BUCKETS), jnp.float32)] * _NUM_HEADS
        row[h] = eT[h]
        e_blocks.append(jnp.concatenate(row, axis=-1))
    e = jnp.concatenate(e_blocks, axis=0)

    e2_blocks = []
    for h in range(_NUM_HEADS):
        row2 = [jnp.zeros((_NUM_BUCKETS, _HEAD_DIM), jnp.bfloat16)] * _NUM_HEADS
        row2[h] = embeddings[h].astype(jnp.bfloat16)
        e2_blocks.append(jnp.concatenate(row2, axis=-1))
    e2 = jnp.concatenate(e2_blocks, axis=0)

    out, loss, uniq = pl.pallas_call(
        _vq_kernel,
        grid=(_G,),
        in_specs=[
            pl.BlockSpec((_R, _EMBED_DIM), lambda i: (i, 0)),
            pl.BlockSpec((1, _EMBED_DIM), lambda i: (0, 0)),
            pl.BlockSpec((1, _EMBED_DIM), lambda i: (0, 0)),
            pl.BlockSpec((_EMBED_DIM, _KDIM), lambda i: (0, 0)),
            pl.BlockSpec((_KDIM, _EMBED_DIM), lambda i: (0, 0)),
        ],
        out_specs=[
            pl.BlockSpec((_R, _EMBED_DIM), lambda i: (i, 0)),
            pl.BlockSpec((1, 1), lambda i: (0, 0)),
            pl.BlockSpec((1, 1), lambda i: (0, 0)),
        ],
        out_shape=[
            jax.ShapeDtypeStruct((_N, _EMBED_DIM), jnp.float32),
            jax.ShapeDtypeStruct((1, 1), jnp.float32),
            jax.ShapeDtypeStruct((1, 1), jnp.int32),
        ],
        scratch_shapes=[
            pltpu.VMEM((1, _KDIM), jnp.float32),
            pltpu.VMEM((1, 1), jnp.float32),
        ],
    )(x, w, b, e, e2)

    quantized_st = out.reshape(_B, _T, _EMBED_DIM)
    return (quantized_st, loss.reshape(()), uniq.reshape(()))


# speedup vs baseline: 1.4782x; 1.0511x over previous
import functools

import jax
import jax.numpy as jnp
from jax.experimental import pallas as pl
from jax.experimental.pallas import tpu as pltpu

_NUM_BUCKETS = 1024
_NUM_HEADS = 4
_EMBED_DIM = 256
_HEAD_DIM = 64
_COMMITMENT_COST = 0.25
_EPSILON = 1e-5
_B, _T = 32, 1024
_N = _B * _T
_R = 1024
_G = _N // _R
_KDIM = _NUM_HEADS * _NUM_BUCKETS


def _vq_kernel(x_ref, w_ref, b_ref, e_ref, e2_ref,
               out_ref, loss_ref, uniq_ref,
               counts_acc, loss_acc):
    step = pl.program_id(0)

    @pl.when(step == 0)
    def _init():
        counts_acc[...] = jnp.zeros_like(counts_acc)
        loss_acc[...] = jnp.zeros_like(loss_acc)

    x = x_ref[...]
    mu = jnp.mean(x, axis=-1, keepdims=True)
    var = jnp.mean((x - mu) ** 2, axis=-1, keepdims=True)
    x = (x - mu) / jnp.sqrt(var + 1e-5) * w_ref[...] + b_ref[...]
    x = jnp.tanh(x / 5.0) * 5.0
    n = jnp.sqrt(jnp.sum(x * x, axis=-1, keepdims=True))
    xn = x / jnp.maximum(n, _EPSILON)

    dist = jnp.dot(xn, e_ref[...], preferred_element_type=jnp.float32)

    masks = []
    for h in range(_NUM_HEADS):
        dh = dist[:, h * _NUM_BUCKETS:(h + 1) * _NUM_BUCKETS]
        mh = jnp.max(dh, axis=-1, keepdims=True)
        masks.append((dh >= mh).astype(jnp.float32))
    mask = jnp.concatenate(masks, axis=-1)

    counts_acc[...] += jnp.sum(mask, axis=0, keepdims=True)

    q = jnp.dot(mask, e2_ref[...], preferred_element_type=jnp.float32)
    out_ref[...] = q

    diff = q - xn
    loss_acc[...] += jnp.sum(diff * diff, axis=(0, 1), keepdims=True)

    @pl.when(step == _G - 1)
    def _fin():
        loss_ref[...] = (_COMMITMENT_COST / (_N * _EMBED_DIM)) * loss_acc[...]
        c = counts_acc[0:1, 0:_NUM_BUCKETS]
        for h in range(1, _NUM_HEADS):
            c = c + counts_acc[0:1, h * _NUM_BUCKETS:(h + 1) * _NUM_BUCKETS]
        uniq_ref[...] = jnp.sum((c > 0.5).astype(jnp.int32),
                                axis=(0, 1), keepdims=True)


@functools.partial(jax.jit, static_argnames=())
def kernel(inputs, ln_weight, ln_bias, embeddings):
    x = inputs.reshape(_N, _EMBED_DIM)
    w = ln_weight.reshape(1, _EMBED_DIM)
    b = ln_bias.reshape(1, _EMBED_DIM)
    eT = jnp.transpose(embeddings, (0, 2, 1))
    e_blocks = []
    for h in range(_NUM_HEADS):
        row = [jnp.zeros((_HEAD_DIM, _NUM_BUCKETS), jnp.float32)] * _NUM_HEADS
        row[h] = eT[h]
        e_blocks.append(jnp.concatenate(row, axis=-1))
    e = jnp.concatenate(e_blocks, axis=0)

    e2_blocks = []
    for h in range(_NUM_HEADS):
        row2 = [jnp.zeros((_NUM_BUCKETS, _HEAD_DIM), jnp.float32)] * _NUM_HEADS
        row2[h] = embeddings[h]
        e2_blocks.append(jnp.concatenate(row2, axis=-1))
    e2 = jnp.concatenate(e2_blocks, axis=0)

    out, loss, uniq = pl.pallas_call(
        _vq_kernel,
        grid=(_G,),
        in_specs=[
            pl.BlockSpec((_R, _EMBED_DIM), lambda i: (i, 0)),
            pl.BlockSpec((1, _EMBED_DIM), lambda i: (0, 0)),
            pl.BlockSpec((1, _EMBED_DIM), lambda i: (0, 0)),
            pl.BlockSpec((_EMBED_DIM, _KDIM), lambda i: (0, 0)),
            pl.BlockSpec((_KDIM, _EMBED_DIM), lambda i: (0, 0)),
        ],
        out_specs=[
            pl.BlockSpec((_R, _EMBED_DIM), lambda i: (i, 0)),
            pl.BlockSpec((1, 1), lambda i: (0, 0)),
            pl.BlockSpec((1, 1), lambda i: (0, 0)),
        ],
        out_shape=[
            jax.ShapeDtypeStruct((_N, _EMBED_DIM), jnp.float32),
            jax.ShapeDtypeStruct((1, 1), jnp.float32),
            jax.ShapeDtypeStruct((1, 1), jnp.int32),
        ],
        scratch_shapes=[
            pltpu.VMEM((1, _KDIM), jnp.float32),
            pltpu.VMEM((1, 1), jnp.float32),
        ],
    )(x, w, b, e, e2)

    quantized_st = out.reshape(_B, _T, _EMBED_DIM)
    return (quantized_st, loss.reshape(()), uniq.reshape(()))
